# baseline (speedup 1.0000x reference)
import functools

import jax
import jax.numpy as jnp
from jax import lax
from jax.experimental import pallas as pl
from jax.experimental.pallas import tpu as pltpu
from jax.experimental.pallas import tpu_sc as plsc

E = 8
D = 768
F = 3072
T = 2048
PAIRS = 2 * T
NC, NS, L = 2, 16, 16
NW = NC * NS
BLK = 256
NB = 24
NP = NB * BLK
PPT = PAIRS // NS
RPT = NP // NW
CH = 48
TPT = T // NW
FC = F // 2


def _router_body(x_ref, wr_ref, br_ref, idx_ref, w_ref):
    x = x_ref[...]
    logits = jnp.dot(x, wr_ref[...], preferred_element_type=jnp.float32)
    logits = logits + br_ref[...]
    m = jnp.max(logits, axis=1, keepdims=True)
    p = jnp.exp(logits - m)
    p = p / jnp.sum(p, axis=1, keepdims=True)
    lane = jax.lax.broadcasted_iota(jnp.int32, p.shape, 1)
    t1 = jnp.max(p, axis=1, keepdims=True)
    i1 = jnp.min(jnp.where(p == t1, lane, E), axis=1, keepdims=True)
    p2 = jnp.where(lane == i1, -1.0, p)
    t2 = jnp.max(p2, axis=1, keepdims=True)
    i2 = jnp.min(jnp.where(p2 == t2, lane, E), axis=1, keepdims=True)
    s = t1 + t2
    idx_ref[...] = jnp.concatenate([i1, i2], axis=1)
    w_ref[...] = jnp.concatenate([t1 / s, t2 / s], axis=1)


def _router(x, W_router, b_router):
    return pl.pallas_call(
        _router_body,
        out_shape=(
            jax.ShapeDtypeStruct((T, 2), jnp.int32),
            jax.ShapeDtypeStruct((T, 2), jnp.float32),
        ),
    )(x, W_router, b_router)


def _dispatch_body(eidx_hbm, w_hbm, x_hbm, xs_hbm, pos_hbm, wsort_hbm,
                   emap_hbm, valid_hbm,
                   eid_v, tok_v, dest_v, dest1_v, wv1_v, wv_v, cnt_v, hist_v,
                   ev_v, zb_v, gidx_v, rows_a, rows_b, hist_sh, srctok_sh,
                   wsort_sh, sem_a, sem_b):
    c = lax.axis_index("c")
    s = lax.axis_index("s")
    wid = c * NS + s
    lane = lax.iota(jnp.int32, L)
    base_pair = s * PPT

    pltpu.sync_copy(eidx_hbm.at[pl.ds(base_pair, PPT)], eid_v)
    cnt = jnp.zeros((L,), jnp.int32)
    for i in range(PPT // L):
        v = eid_v[pl.ds(i * L, L)]
        for e in range(E):
            pc = jnp.sum((v == e).astype(jnp.int32))
            cnt = cnt + jnp.where(lane == e, pc, 0)
    cnt_v[...] = cnt
    pltpu.sync_copy(cnt_v, hist_sh.at[pl.ds(s * L, L)])

    for i in range(0, NP // NS, L):
        zb_v[pl.ds(i, L)] = jnp.zeros((L,), jnp.int32)
    plsc.subcore_barrier()
    pltpu.sync_copy(zb_v, srctok_sh.at[pl.ds(s * (NP // NS), NP // NS)])

    pltpu.sync_copy(hist_sh, hist_v)
    total = jnp.zeros((L,), jnp.int32)
    prior = jnp.zeros((L,), jnp.int32)
    for j in range(NS):
        row = hist_v[pl.ds(j * L, L)]
        total = total + row
        prior = prior + jnp.where(j < s, row, 0)
    blocks = (total + (BLK - 1)) // BLK
    bstart = plsc.cumsum(blocks) - blocks
    mybase = bstart * BLK + prior
    run = [jnp.sum(jnp.where(lane == e, mybase, 0)) for e in range(E)]

    for i in range(PPT // L):
        v = eid_v[pl.ds(i * L, L)]
        dest = jnp.zeros((L,), jnp.int32)
        for e in range(E):
            msk = v == e
            mi = msk.astype(jnp.int32)
            rank = plsc.cumsum(mi) - 1
            dest = jnp.where(msk, run[e] + rank, dest)
            run[e] = run[e] + jnp.sum(mi)
        dest_v[i // 8, pl.ds((i % 8) * L, L)] = dest
        dest1_v[pl.ds(i * L, L)] = dest
        tok_v[i // 8, pl.ds((i % 8) * L, L)] = (base_pair + i * L + lane) >> 1

    pltpu.sync_copy(w_hbm.at[pl.ds(base_pair, PPT)], wv1_v)
    for i in range(PPT // L):
        wv_v[i // 8, pl.ds((i % 8) * L, L)] = wv1_v[pl.ds(i * L, L)]
    plsc.subcore_barrier()
    for j in range(PPT // 128):
        pltpu.sync_copy(tok_v.at[j], srctok_sh.at[dest_v.at[j]])

    @pl.when(c == 0)
    def _scatter_w():
        for j in range(PPT // 128):
            pltpu.sync_copy(wv_v.at[j], wsort_sh.at[dest_v.at[j]])

    plsc.subcore_barrier()

    @pl.when(c == 0)
    def _emit_pos():
        pltpu.sync_copy(dest1_v, pos_hbm.at[pl.ds(base_pair, PPT)])
        pltpu.sync_copy(wsort_sh.at[pl.ds(s * (NP // NS), NP // NS)],
                        wsort_hbm.at[pl.ds(s * (NP // NS), NP // NS)])

    @pl.when((c == 0) & (s == 0))
    def _emit_emap():
        nblk = jnp.sum(blocks)
        lastused = jnp.max(jnp.where(blocks > 0, lane, -1))
        bst = [jnp.sum(jnp.where(lane == e, bstart, 0)) for e in range(E)]
        for chunk in range(2):
            bvec = lax.iota(jnp.int32, L) + chunk * L
            owner = jnp.full((L,), -1, jnp.int32)
            for e in range(E):
                owner = owner + (bvec >= bst[e]).astype(jnp.int32)
            owner = jnp.where(bvec < nblk, owner, lastused)
            ev_v[pl.ds(chunk * L, L)] = owner
        pltpu.sync_copy(ev_v, emap_hbm)
        for chunk in range(2):
            bvec = lax.iota(jnp.int32, L) + chunk * L
            ev_v[pl.ds(chunk * L, L)] = (bvec < nblk).astype(jnp.int32)
        pltpu.sync_copy(ev_v, valid_hbm)

    rbase = wid * RPT
    pltpu.sync_copy(srctok_sh.at[pl.ds(rbase, RPT)], gidx_v)
    bufs = (rows_a, rows_b)
    sems = (sem_a, sem_b)
    nch = RPT // CH
    descs = [None] * nch

    def start(i):
        descs[i] = pltpu.async_copy(
            x_hbm.at[gidx_v.at[pl.ds(i * CH, CH)]], bufs[i % 2], sems[i % 2])

    start(0)
    for i in range(nch):
        descs[i].wait()
        if i + 1 < nch:
            start(i + 1)
        pltpu.sync_copy(bufs[i % 2], xs_hbm.at[pl.ds(rbase + i * CH, CH)])


def _dispatch_gather(eidx_flat, wflat2, x):
    mesh = plsc.VectorSubcoreMesh(core_axis_name="c", subcore_axis_name="s",
                                  num_cores=NC, num_subcores=NS)
    f = pl.kernel(
        _dispatch_body,
        out_type=(
            jax.ShapeDtypeStruct((NP, D), jnp.float32),
            jax.ShapeDtypeStruct((PAIRS,), jnp.int32),
            jax.ShapeDtypeStruct((NP,), jnp.float32),
            jax.ShapeDtypeStruct((2 * L,), jnp.int32),
            jax.ShapeDtypeStruct((2 * L,), jnp.int32),
        ),
        mesh=mesh,
        compiler_params=pltpu.CompilerParams(needs_layout_passes=False),
        scratch_types=[
            pltpu.VMEM((PPT,), jnp.int32),
            pltpu.VMEM((PPT // 128, 128), jnp.int32),
            pltpu.VMEM((PPT // 128, 128), jnp.int32),
            pltpu.VMEM((PPT,), jnp.int32),
            pltpu.VMEM((PPT,), jnp.float32),
            pltpu.VMEM((PPT // 128, 128), jnp.float32),
            pltpu.VMEM((L,), jnp.int32),
            pltpu.VMEM((NS * L,), jnp.int32),
            pltpu.VMEM((2 * L,), jnp.int32),
            pltpu.VMEM((NP // NS,), jnp.int32),
            pltpu.VMEM((RPT,), jnp.int32),
            pltpu.VMEM((CH, D), jnp.float32),
            pltpu.VMEM((CH, D), jnp.float32),
            pltpu.VMEM_SHARED((NS * L,), jnp.int32),
            pltpu.VMEM_SHARED((NP,), jnp.int32),
            pltpu.VMEM_SHARED((NP,), jnp.float32),
            pltpu.SemaphoreType.DMA,
            pltpu.SemaphoreType.DMA,
        ],
    )
    return f(eidx_flat, wflat2, x)


def _group_body(emap_ref, valid_ref, xs_ref, wcol_ref, win_ref, bin_ref,
                wout_ref, bout_ref, out_ref):
    b = pl.program_id(0)
    fc = pl.program_id(1)

    @pl.when(valid_ref[b] > 0)
    def _():
        xb = xs_ref[...].astype(jnp.bfloat16)
        h = jnp.dot(xb, win_ref[0].astype(jnp.bfloat16),
                    preferred_element_type=jnp.float32)
        h = h + bin_ref[0]
        h = jax.nn.gelu(h, approximate=True)
        y = jnp.dot(h.astype(jnp.bfloat16), wout_ref[0].astype(jnp.bfloat16),
                    preferred_element_type=jnp.float32)
        wcol = wcol_ref[...]

        @pl.when(fc == 0)
        def _init():
            out_ref[...] = wcol * (y + bout_ref[0])

        @pl.when(fc != 0)
        def _acc():
            out_ref[...] += wcol * y


def _grouped_mlp(emap, valid, xs, wcol, W_in, b_in3, W_out, b_out3):
    grid_spec = pltpu.PrefetchScalarGridSpec(
        num_scalar_prefetch=2,
        grid=(NB, 2),
        in_specs=[
            pl.BlockSpec((BLK, D), lambda b, f, emap, valid: (b, 0)),
            pl.BlockSpec((BLK, 1), lambda b, f, emap, valid: (b, 0)),
            pl.BlockSpec((1, D, FC), lambda b, f, emap, valid: (emap[b], 0, f)),
            pl.BlockSpec((1, 1, FC), lambda b, f, emap, valid: (emap[b], 0, f)),
            pl.BlockSpec((1, FC, D), lambda b, f, emap, valid: (emap[b], f, 0)),
            pl.BlockSpec((1, 1, D), lambda b, f, emap, valid: (emap[b], 0, 0)),
        ],
        out_specs=pl.BlockSpec((BLK, D), lambda b, f, emap, valid: (b, 0)),
    )
    return pl.pallas_call(
        _group_body,
        grid_spec=grid_spec,
        out_shape=jax.ShapeDtypeStruct((NP, D), jnp.float32),
        compiler_params=pltpu.CompilerParams(
            dimension_semantics=("arbitrary", "arbitrary"),
        ),
    )(emap, valid, xs, wcol, W_in, b_in3, W_out, b_out3)


def _combine_body(pos_hbm, ys_hbm, out_hbm, pos_v, rows_v, ob_v, sem):
    c = lax.axis_index("c")
    s = lax.axis_index("s")
    wid = c * NS + s
    tbase = wid * TPT
    pltpu.sync_copy(pos_hbm.at[pl.ds(wid * 2 * TPT, 2 * TPT)], pos_v)
    for half in range(2):
        pltpu.async_copy(ys_hbm.at[pos_v.at[pl.ds(half * 64, 64)]], rows_v,
                         sem).wait()

        def body(t, _):
            for j in range(D // L):
                r0 = rows_v[2 * t, pl.ds(j * L, L)]
                r1 = rows_v[2 * t + 1, pl.ds(j * L, L)]
                ob_v[t, pl.ds(j * L, L)] = r0 + r1
            return 0

        lax.fori_loop(0, TPT // 2, body, 0)
        pltpu.sync_copy(ob_v, out_hbm.at[pl.ds(tbase + half * (TPT // 2),
                                               TPT // 2)])


def _combine(pos, ys):
    mesh = plsc.VectorSubcoreMesh(core_axis_name="c", subcore_axis_name="s",
                                  num_cores=NC, num_subcores=NS)
    f = pl.kernel(
        _combine_body,
        out_type=jax.ShapeDtypeStruct((T, D), jnp.float32),
        name="moe_combine",
        mesh=mesh,
        compiler_params=pltpu.CompilerParams(needs_layout_passes=False),
        scratch_types=[
            pltpu.VMEM((2 * TPT,), jnp.int32),
            pltpu.VMEM((TPT, D), jnp.float32),
            pltpu.VMEM((TPT // 2, D), jnp.float32),
            pltpu.SemaphoreType.DMA,
        ],
    )
    return f(pos, ys)


def kernel(residual, W_router, b_router, W_in, b_in, W_out, b_out):
    Bt, St, Dm = residual.shape
    x = residual.reshape(T, D)
    idx, w = _router(x, W_router, b_router.reshape(1, E))
    xs, pos, wsort, emap, valid = _dispatch_gather(
        idx.reshape(PAIRS), w.reshape(PAIRS), x)
    ys = _grouped_mlp(emap, valid, xs, wsort.reshape(NP, 1), W_in,
                      b_in.reshape(E, 1, F), W_out, b_out.reshape(E, 1, D))
    out = _combine(pos, ys)
    return out.reshape(Bt, St, Dm)


# speedup vs baseline: 1.4100x; 1.4100x over previous
import functools

import jax
import jax.numpy as jnp
from jax import lax
from jax.experimental import pallas as pl
from jax.experimental.pallas import tpu as pltpu
from jax.experimental.pallas import tpu_sc as plsc

E = 8
D = 768
F = 3072
T = 2048
PAIRS = 2 * T
NC, NS, L = 2, 16, 16
NW = NC * NS
BLK = 256
NB = 24
NP = NB * BLK
PPT = PAIRS // NS
RPT = NP // NW
CH = 48
TPT = T // NW
FC = F // 2


def _router_body(x_ref, wr_ref, br_ref, idx_ref, w_ref):
    x = x_ref[...]
    logits = jnp.dot(x, wr_ref[...], preferred_element_type=jnp.float32)
    logits = logits + br_ref[...]
    m = jnp.max(logits, axis=1, keepdims=True)
    p = jnp.exp(logits - m)
    p = p / jnp.sum(p, axis=1, keepdims=True)
    lane = jax.lax.broadcasted_iota(jnp.int32, p.shape, 1)
    t1 = jnp.max(p, axis=1, keepdims=True)
    i1 = jnp.min(jnp.where(p == t1, lane, E), axis=1, keepdims=True)
    p2 = jnp.where(lane == i1, -1.0, p)
    t2 = jnp.max(p2, axis=1, keepdims=True)
    i2 = jnp.min(jnp.where(p2 == t2, lane, E), axis=1, keepdims=True)
    s = t1 + t2
    idx_ref[...] = jnp.concatenate([i1, i2], axis=1)
    w_ref[...] = jnp.concatenate([t1 / s, t2 / s], axis=1)


def _router(x, W_router, b_router):
    return pl.pallas_call(
        _router_body,
        out_shape=(
            jax.ShapeDtypeStruct((T, 2), jnp.int32),
            jax.ShapeDtypeStruct((T, 2), jnp.float32),
        ),
    )(x, W_router, b_router)


def _dispatch_body(eidx_hbm, w_hbm, x_hbm, xs_hbm, pos_hbm, wsort_hbm,
                   emap_hbm, valid_hbm,
                   eid_v, dest_v, dest1_v, qd0_v, qd1_v, qd2_v, qd3_v, wv1_v,
                   wv_v, cnt_v, hist_v, ev_v, rows_a, rows_b, hist_sh,
                   sem_a, sem_b):
    c = lax.axis_index("c")
    s = lax.axis_index("s")
    lane = lax.iota(jnp.int32, L)
    base_pair = s * PPT

    pltpu.sync_copy(eidx_hbm.at[pl.ds(base_pair, PPT)], eid_v)
    cnt = jnp.zeros((L,), jnp.int32)
    for i in range(PPT // L):
        v = eid_v[pl.ds(i * L, L)]
        for e in range(E):
            pc = jnp.sum((v == e).astype(jnp.int32))
            cnt = cnt + jnp.where(lane == e, pc, 0)
    cnt_v[...] = cnt
    pltpu.sync_copy(cnt_v, hist_sh.at[pl.ds(s * L, L)])
    pltpu.sync_copy(w_hbm.at[pl.ds(base_pair, PPT)], wv1_v)
    plsc.subcore_barrier()

    pltpu.sync_copy(hist_sh, hist_v)
    total = jnp.zeros((L,), jnp.int32)
    prior = jnp.zeros((L,), jnp.int32)
    for j in range(NS):
        row = hist_v[pl.ds(j * L, L)]
        total = total + row
        prior = prior + jnp.where(j < s, row, 0)
    blocks = (total + (BLK - 1)) // BLK
    bstart = plsc.cumsum(blocks) - blocks
    mybase = bstart * BLK + prior
    run = [jnp.sum(jnp.where(lane == e, mybase, 0)) for e in range(E)]

    quarters = (qd0_v, qd1_v, qd2_v, qd3_v)
    for i in range(PPT // L):
        v = eid_v[pl.ds(i * L, L)]
        dest = jnp.zeros((L,), jnp.int32)
        for e in range(E):
            msk = v == e
            mi = msk.astype(jnp.int32)
            rank = plsc.cumsum(mi) - 1
            dest = jnp.where(msk, run[e] + rank, dest)
            run[e] = run[e] + jnp.sum(mi)
        dest_v[i // 8, pl.ds((i % 8) * L, L)] = dest
        dest1_v[pl.ds(i * L, L)] = dest
        quarters[i // 4][pl.ds((i % 4) * L, L)] = dest
        wv_v[i // 8, pl.ds((i % 8) * L, L)] = wv1_v[pl.ds(i * L, L)]

    @pl.when(c == 0)
    def _emit_pos():
        pltpu.sync_copy(dest1_v, pos_hbm.at[pl.ds(base_pair, PPT)])
        for j in range(PPT // 128):
            pltpu.sync_copy(wv_v.at[j], wsort_hbm.at[dest_v.at[j]])

    @pl.when((c == 0) & (s == 0))
    def _emit_emap():
        nblk = jnp.sum(blocks)
        lastused = jnp.max(jnp.where(blocks > 0, lane, -1))
        bst = [jnp.sum(jnp.where(lane == e, bstart, 0)) for e in range(E)]
        for chunk in range(2):
            bvec = lax.iota(jnp.int32, L) + chunk * L
            owner = jnp.full((L,), -1, jnp.int32)
            for e in range(E):
                owner = owner + (bvec >= bst[e]).astype(jnp.int32)
            owner = jnp.where(bvec < nblk, owner, lastused)
            ev_v[pl.ds(chunk * L, L)] = owner
        pltpu.sync_copy(ev_v, emap_hbm)
        for chunk in range(2):
            bvec = lax.iota(jnp.int32, L) + chunk * L
            ev_v[pl.ds(chunk * L, L)] = (bvec < nblk).astype(jnp.int32)
        pltpu.sync_copy(ev_v, valid_hbm)

    tok0 = (s % 8) * PPT

    def do_quarter(qidx, buf, sem):
        pltpu.sync_copy(x_hbm.at[pl.ds(tok0 + qidx * 64, 64)], buf)
        return pltpu.async_copy(buf, xs_hbm.at[quarters[qidx]], sem)

    @pl.when(c == 0)
    def _scatter01():
        d0 = do_quarter(0, rows_a, sem_a)
        d1 = do_quarter(1, rows_b, sem_b)
        d0.wait()
        d1.wait()

    @pl.when(c == 1)
    def _scatter23():
        d2 = do_quarter(2, rows_a, sem_a)
        d3 = do_quarter(3, rows_b, sem_b)
        d2.wait()
        d3.wait()


def _dispatch_gather(eidx_flat, wflat2, x):
    mesh = plsc.VectorSubcoreMesh(core_axis_name="c", subcore_axis_name="s",
                                  num_cores=NC, num_subcores=NS)
    f = pl.kernel(
        _dispatch_body,
        out_type=(
            jax.ShapeDtypeStruct((NP, D), jnp.float32),
            jax.ShapeDtypeStruct((PAIRS,), jnp.int32),
            jax.ShapeDtypeStruct((NP,), jnp.float32),
            jax.ShapeDtypeStruct((2 * L,), jnp.int32),
            jax.ShapeDtypeStruct((2 * L,), jnp.int32),
        ),
        mesh=mesh,
        compiler_params=pltpu.CompilerParams(needs_layout_passes=False),
        scratch_types=[
            pltpu.VMEM((PPT,), jnp.int32),
            pltpu.VMEM((PPT // 128, 128), jnp.int32),
            pltpu.VMEM((PPT,), jnp.int32),
            pltpu.VMEM((64,), jnp.int32),
            pltpu.VMEM((64,), jnp.int32),
            pltpu.VMEM((64,), jnp.int32),
            pltpu.VMEM((64,), jnp.int32),
            pltpu.VMEM((PPT,), jnp.float32),
            pltpu.VMEM((PPT // 128, 128), jnp.float32),
            pltpu.VMEM((L,), jnp.int32),
            pltpu.VMEM((NS * L,), jnp.int32),
            pltpu.VMEM((2 * L,), jnp.int32),
            pltpu.VMEM((64, D), jnp.float32),
            pltpu.VMEM((64, D), jnp.float32),
            pltpu.VMEM_SHARED((NS * L,), jnp.int32),
            pltpu.SemaphoreType.DMA,
            pltpu.SemaphoreType.DMA,
        ],
    )
    return f(eidx_flat, wflat2, x)


def _group_body(emap_ref, valid_ref, xs_ref, wcol_ref, win_ref, bin_ref,
                wout_ref, bout_ref, out_ref):
    b = pl.program_id(0)
    fc = pl.program_id(1)

    @pl.when(valid_ref[b] > 0)
    def _():
        xb = xs_ref[...].astype(jnp.bfloat16)
        h = jnp.dot(xb, win_ref[0].astype(jnp.bfloat16),
                    preferred_element_type=jnp.float32)
        h = h + bin_ref[0]
        h = jax.nn.gelu(h, approximate=True)
        y = jnp.dot(h.astype(jnp.bfloat16), wout_ref[0].astype(jnp.bfloat16),
                    preferred_element_type=jnp.float32)
        wcol = wcol_ref[...]

        @pl.when(fc == 0)
        def _init():
            out_ref[...] = wcol * (y + bout_ref[0])

        @pl.when(fc != 0)
        def _acc():
            out_ref[...] += wcol * y


def _grouped_mlp(emap, valid, xs, wcol, W_in, b_in3, W_out, b_out3):
    grid_spec = pltpu.PrefetchScalarGridSpec(
        num_scalar_prefetch=2,
        grid=(NB, 2),
        in_specs=[
            pl.BlockSpec((BLK, D), lambda b, f, emap, valid: (b, 0)),
            pl.BlockSpec((BLK, 1), lambda b, f, emap, valid: (b, 0)),
            pl.BlockSpec((1, D, FC), lambda b, f, emap, valid: (emap[b], 0, f)),
            pl.BlockSpec((1, 1, FC), lambda b, f, emap, valid: (emap[b], 0, f)),
            pl.BlockSpec((1, FC, D), lambda b, f, emap, valid: (emap[b], f, 0)),
            pl.BlockSpec((1, 1, D), lambda b, f, emap, valid: (emap[b], 0, 0)),
        ],
        out_specs=pl.BlockSpec((BLK, D), lambda b, f, emap, valid: (b, 0)),
    )
    return pl.pallas_call(
        _group_body,
        grid_spec=grid_spec,
        out_shape=jax.ShapeDtypeStruct((NP, D), jnp.float32),
        compiler_params=pltpu.CompilerParams(
            dimension_semantics=("arbitrary", "arbitrary"),
        ),
    )(emap, valid, xs, wcol, W_in, b_in3, W_out, b_out3)


def _combine_body(pos_hbm, ys_hbm, out_hbm, pos0_v, pos1_v, rows0_v, rows1_v,
                  ob_v, sem0, sem1):
    c = lax.axis_index("c")
    s = lax.axis_index("s")
    wid = c * NS + s
    tbase = wid * TPT
    pltpu.sync_copy(pos_hbm.at[pl.ds(tbase, TPT)], pos0_v)
    pltpu.sync_copy(pos_hbm.at[pl.ds(T + tbase, TPT)], pos1_v)
    H = TPT // 2
    for half in range(2):
        d0 = pltpu.async_copy(ys_hbm.at[pos0_v.at[pl.ds(half * H, H)]],
                              rows0_v, sem0)
        d1 = pltpu.async_copy(ys_hbm.at[pos1_v.at[pl.ds(half * H, H)]],
                              rows1_v, sem1)
        d0.wait()
        d1.wait()

        def body(t, _):
            for j in range(D // L):
                r0 = rows0_v[t, pl.ds(j * L, L)]
                r1 = rows1_v[t, pl.ds(j * L, L)]
                ob_v[t, pl.ds(j * L, L)] = r0 + r1
            return 0

        lax.fori_loop(0, H, body, 0)
        pltpu.sync_copy(ob_v, out_hbm.at[pl.ds(tbase + half * H, H)])


def _combine(pos, ys):
    mesh = plsc.VectorSubcoreMesh(core_axis_name="c", subcore_axis_name="s",
                                  num_cores=NC, num_subcores=NS)
    f = pl.kernel(
        _combine_body,
        out_type=jax.ShapeDtypeStruct((T, D), jnp.float32),
        name="moe_combine",
        mesh=mesh,
        compiler_params=pltpu.CompilerParams(needs_layout_passes=False),
        scratch_types=[
            pltpu.VMEM((TPT,), jnp.int32),
            pltpu.VMEM((TPT,), jnp.int32),
            pltpu.VMEM((TPT // 2, D), jnp.float32),
            pltpu.VMEM((TPT // 2, D), jnp.float32),
            pltpu.VMEM((TPT // 2, D), jnp.float32),
            pltpu.SemaphoreType.DMA,
            pltpu.SemaphoreType.DMA,
        ],
    )
    return f(pos, ys)


def kernel(residual, W_router, b_router, W_in, b_in, W_out, b_out):
    Bt, St, Dm = residual.shape
    x = residual.reshape(T, D)
    idx, w = _router(x, W_router, b_router.reshape(1, E))
    xs, pos, wsort, emap, valid = _dispatch_gather(
        idx.T.reshape(PAIRS), w.T.reshape(PAIRS), x)
    ys = _grouped_mlp(emap, valid, xs, wsort.reshape(NP, 1), W_in,
                      b_in.reshape(E, 1, F), W_out, b_out.reshape(E, 1, D))
    out = _combine(pos, ys)
    return out.reshape(Bt, St, Dm)


# speedup vs baseline: 1.7630x; 1.2503x over previous
import functools

import jax
import jax.numpy as jnp
from jax import lax
from jax.experimental import pallas as pl
from jax.experimental.pallas import tpu as pltpu
from jax.experimental.pallas import tpu_sc as plsc

E = 8
D = 768
F = 3072
T = 2048
PAIRS = 2 * T
NC, NS, L = 2, 16, 16
NW = NC * NS
BLK = 256
NB = 24
NP = NB * BLK
PPT = PAIRS // NS
RPT = NP // NW
CH = 48
TPT = T // NW
FC = F // 2


def _router_body(x_ref, wr_ref, br_ref, idx_ref, w_ref):
    x = x_ref[...]
    logits = jnp.dot(x, wr_ref[...], preferred_element_type=jnp.float32)
    logits = logits + br_ref[...]
    m = jnp.max(logits, axis=1, keepdims=True)
    p = jnp.exp(logits - m)
    p = p / jnp.sum(p, axis=1, keepdims=True)
    lane = jax.lax.broadcasted_iota(jnp.int32, p.shape, 1)
    t1 = jnp.max(p, axis=1, keepdims=True)
    i1 = jnp.min(jnp.where(p == t1, lane, E), axis=1, keepdims=True)
    p2 = jnp.where(lane == i1, -1.0, p)
    t2 = jnp.max(p2, axis=1, keepdims=True)
    i2 = jnp.min(jnp.where(p2 == t2, lane, E), axis=1, keepdims=True)
    s = t1 + t2
    idx_ref[...] = jnp.concatenate([i1, i2], axis=1)
    w_ref[...] = jnp.concatenate([t1 / s, t2 / s], axis=1)


def _router(x, W_router, b_router):
    return pl.pallas_call(
        _router_body,
        out_shape=(
            jax.ShapeDtypeStruct((T, 2), jnp.int32),
            jax.ShapeDtypeStruct((T, 2), jnp.float32),
        ),
    )(x, W_router, b_router)


def _dispatch_body(eidx_hbm, w_hbm, x_hbm, xs_hbm, pos_hbm, wsort_hbm,
                   emap_hbm, valid_hbm,
                   eid_v, dest_v, dest1_v, qd0_v, qd1_v, qd2_v, qd3_v, wv1_v,
                   wv_v, cnt_v, hist_v, ev_v, rows_a, rows_b, hist_sh,
                   sem_a, sem_b):
    c = lax.axis_index("c")
    s = lax.axis_index("s")
    lane = lax.iota(jnp.int32, L)
    base_pair = s * PPT

    pltpu.sync_copy(eidx_hbm.at[pl.ds(base_pair, PPT)], eid_v)
    cnt = jnp.zeros((L,), jnp.int32)
    for i in range(PPT // L):
        v = eid_v[pl.ds(i * L, L)]
        for e in range(E):
            pc = jnp.sum((v == e).astype(jnp.int32))
            cnt = cnt + jnp.where(lane == e, pc, 0)
    cnt_v[...] = cnt
    pltpu.sync_copy(cnt_v, hist_sh.at[pl.ds(s * L, L)])
    pltpu.sync_copy(w_hbm.at[pl.ds(base_pair, PPT)], wv1_v)
    plsc.subcore_barrier()

    pltpu.sync_copy(hist_sh, hist_v)
    total = jnp.zeros((L,), jnp.int32)
    prior = jnp.zeros((L,), jnp.int32)
    for j in range(NS):
        row = hist_v[pl.ds(j * L, L)]
        total = total + row
        prior = prior + jnp.where(j < s, row, 0)
    blocks = (total + (BLK - 1)) // BLK
    bstart = plsc.cumsum(blocks) - blocks
    mybase = bstart * BLK + prior
    run = [jnp.sum(jnp.where(lane == e, mybase, 0)) for e in range(E)]

    quarters = (qd0_v, qd1_v, qd2_v, qd3_v)
    for i in range(PPT // L):
        v = eid_v[pl.ds(i * L, L)]
        dest = jnp.zeros((L,), jnp.int32)
        for e in range(E):
            msk = v == e
            mi = msk.astype(jnp.int32)
            rank = plsc.cumsum(mi) - 1
            dest = jnp.where(msk, run[e] + rank, dest)
            run[e] = run[e] + jnp.sum(mi)
        dest_v[i // 8, pl.ds((i % 8) * L, L)] = dest
        dest1_v[pl.ds(i * L, L)] = dest
        quarters[i // 4][pl.ds((i % 4) * L, L)] = dest
        wv_v[i // 8, pl.ds((i % 8) * L, L)] = wv1_v[pl.ds(i * L, L)]

    @pl.when(c == 0)
    def _emit_pos():
        pltpu.sync_copy(dest1_v, pos_hbm.at[pl.ds(base_pair, PPT)])
        for j in range(PPT // 128):
            pltpu.sync_copy(wv_v.at[j], wsort_hbm.at[dest_v.at[j]])

    @pl.when((c == 0) & (s == 0))
    def _emit_emap():
        nblk = jnp.sum(blocks)
        lastused = jnp.max(jnp.where(blocks > 0, lane, -1))
        bst = [jnp.sum(jnp.where(lane == e, bstart, 0)) for e in range(E)]
        for chunk in range(2):
            bvec = lax.iota(jnp.int32, L) + chunk * L
            owner = jnp.full((L,), -1, jnp.int32)
            for e in range(E):
                owner = owner + (bvec >= bst[e]).astype(jnp.int32)
            owner = jnp.where(bvec < nblk, owner, lastused)
            ev_v[pl.ds(chunk * L, L)] = owner
        pltpu.sync_copy(ev_v, emap_hbm)
        for chunk in range(2):
            bvec = lax.iota(jnp.int32, L) + chunk * L
            ev_v[pl.ds(chunk * L, L)] = (bvec < nblk).astype(jnp.int32)
        pltpu.sync_copy(ev_v, valid_hbm)

    tok0 = (s % 8) * PPT

    def do_quarter(qidx, buf, sem):
        pltpu.sync_copy(x_hbm.at[pl.ds(tok0 + qidx * 64, 64)], buf)
        return pltpu.async_copy(buf, xs_hbm.at[quarters[qidx]], sem)

    @pl.when(c == 0)
    def _scatter01():
        d0 = do_quarter(0, rows_a, sem_a)
        d1 = do_quarter(1, rows_b, sem_b)
        d0.wait()
        d1.wait()

    @pl.when(c == 1)
    def _scatter23():
        d2 = do_quarter(2, rows_a, sem_a)
        d3 = do_quarter(3, rows_b, sem_b)
        d2.wait()
        d3.wait()


def _dispatch_gather(eidx_flat, wflat2, x):
    mesh = plsc.VectorSubcoreMesh(core_axis_name="c", subcore_axis_name="s",
                                  num_cores=NC, num_subcores=NS)
    f = pl.kernel(
        _dispatch_body,
        out_type=(
            jax.ShapeDtypeStruct((NP, D), jnp.float32),
            jax.ShapeDtypeStruct((PAIRS,), jnp.int32),
            jax.ShapeDtypeStruct((NP,), jnp.float32),
            jax.ShapeDtypeStruct((2 * L,), jnp.int32),
            jax.ShapeDtypeStruct((2 * L,), jnp.int32),
        ),
        mesh=mesh,
        compiler_params=pltpu.CompilerParams(needs_layout_passes=False),
        scratch_types=[
            pltpu.VMEM((PPT,), jnp.int32),
            pltpu.VMEM((PPT // 128, 128), jnp.int32),
            pltpu.VMEM((PPT,), jnp.int32),
            pltpu.VMEM((64,), jnp.int32),
            pltpu.VMEM((64,), jnp.int32),
            pltpu.VMEM((64,), jnp.int32),
            pltpu.VMEM((64,), jnp.int32),
            pltpu.VMEM((PPT,), jnp.float32),
            pltpu.VMEM((PPT // 128, 128), jnp.float32),
            pltpu.VMEM((L,), jnp.int32),
            pltpu.VMEM((NS * L,), jnp.int32),
            pltpu.VMEM((2 * L,), jnp.int32),
            pltpu.VMEM((64, D), jnp.float32),
            pltpu.VMEM((64, D), jnp.float32),
            pltpu.VMEM_SHARED((NS * L,), jnp.int32),
            pltpu.SemaphoreType.DMA,
            pltpu.SemaphoreType.DMA,
        ],
    )
    return f(eidx_flat, wflat2, x)


def _group_body(emap_ref, valid_ref, xs_ref, wcol_ref, win_ref, bin_ref,
                wout_ref, bout_ref, out_ref):
    b = pl.program_id(0)

    @pl.when(valid_ref[b] > 0)
    def _():
        xb = xs_ref[...].astype(jnp.bfloat16)
        h = jnp.dot(xb, win_ref[0].astype(jnp.bfloat16),
                    preferred_element_type=jnp.float32)
        h = h + bin_ref[0]
        h = jax.nn.gelu(h, approximate=True)
        y = jnp.dot(h.astype(jnp.bfloat16), wout_ref[0].astype(jnp.bfloat16),
                    preferred_element_type=jnp.float32)
        out_ref[...] = wcol_ref[...] * (y + bout_ref[0])


def _grouped_mlp(emap, valid, xs, wcol, W_in, b_in3, W_out, b_out3):
    grid_spec = pltpu.PrefetchScalarGridSpec(
        num_scalar_prefetch=2,
        grid=(NB,),
        in_specs=[
            pl.BlockSpec((BLK, D), lambda b, emap, valid: (b, 0)),
            pl.BlockSpec((BLK, 1), lambda b, emap, valid: (b, 0)),
            pl.BlockSpec((1, D, F), lambda b, emap, valid: (emap[b], 0, 0)),
            pl.BlockSpec((1, 1, F), lambda b, emap, valid: (emap[b], 0, 0)),
            pl.BlockSpec((1, F, D), lambda b, emap, valid: (emap[b], 0, 0)),
            pl.BlockSpec((1, 1, D), lambda b, emap, valid: (emap[b], 0, 0)),
        ],
        out_specs=pl.BlockSpec((BLK, D), lambda b, emap, valid: (b, 0)),
    )
    return pl.pallas_call(
        _group_body,
        grid_spec=grid_spec,
        out_shape=jax.ShapeDtypeStruct((NP, D), jnp.float32),
        compiler_params=pltpu.CompilerParams(
            dimension_semantics=("arbitrary",),
        ),
    )(emap, valid, xs, wcol, W_in, b_in3, W_out, b_out3)


def _combine_body(pos_hbm, ys_hbm, out_hbm, pos0_v, pos1_v, rows0_v, rows1_v,
                  ob_v, sem0, sem1):
    c = lax.axis_index("c")
    s = lax.axis_index("s")
    wid = c * NS + s
    tbase = wid * TPT
    pltpu.sync_copy(pos_hbm.at[pl.ds(tbase, TPT)], pos0_v)
    pltpu.sync_copy(pos_hbm.at[pl.ds(T + tbase, TPT)], pos1_v)
    H = TPT // 2
    for half in range(2):
        d0 = pltpu.async_copy(ys_hbm.at[pos0_v.at[pl.ds(half * H, H)]],
                              rows0_v, sem0)
        d1 = pltpu.async_copy(ys_hbm.at[pos1_v.at[pl.ds(half * H, H)]],
                              rows1_v, sem1)
        d0.wait()
        d1.wait()

        def body(t, _):
            for j in range(D // L):
                r0 = rows0_v[t, pl.ds(j * L, L)]
                r1 = rows1_v[t, pl.ds(j * L, L)]
                ob_v[t, pl.ds(j * L, L)] = r0 + r1
            return 0

        lax.fori_loop(0, H, body, 0)
        pltpu.sync_copy(ob_v, out_hbm.at[pl.ds(tbase + half * H, H)])


def _combine(pos, ys):
    mesh = plsc.VectorSubcoreMesh(core_axis_name="c", subcore_axis_name="s",
                                  num_cores=NC, num_subcores=NS)
    f = pl.kernel(
        _combine_body,
        out_type=jax.ShapeDtypeStruct((T, D), jnp.float32),
        name="moe_combine",
        mesh=mesh,
        compiler_params=pltpu.CompilerParams(needs_layout_passes=False),
        scratch_types=[
            pltpu.VMEM((TPT,), jnp.int32),
            pltpu.VMEM((TPT,), jnp.int32),
            pltpu.VMEM((TPT // 2, D), jnp.float32),
            pltpu.VMEM((TPT // 2, D), jnp.float32),
            pltpu.VMEM((TPT // 2, D), jnp.float32),
            pltpu.SemaphoreType.DMA,
            pltpu.SemaphoreType.DMA,
        ],
    )
    return f(pos, ys)


def kernel(residual, W_router, b_router, W_in, b_in, W_out, b_out):
    Bt, St, Dm = residual.shape
    x = residual.reshape(T, D)
    idx, w = _router(x, W_router, b_router.reshape(1, E))
    xs, pos, wsort, emap, valid = _dispatch_gather(
        idx.T.reshape(PAIRS), w.T.reshape(PAIRS), x)
    ys = _grouped_mlp(emap, valid, xs, wsort.reshape(NP, 1), W_in,
                      b_in.reshape(E, 1, F), W_out, b_out.reshape(E, 1, D))
    out = _combine(pos, ys)
    return out.reshape(Bt, St, Dm)
